# baseline (speedup 1.0000x reference)
import functools
import jax
import jax.numpy as jnp
from jax import lax
from jax.experimental import pallas as pl
from jax.experimental.pallas import tpu as pltpu
from jax.experimental.pallas import tpu_sc as plsc

_K = 8
_L = 16
_CHUNK = 512
_RU = 4


def _sc_kernel_body(B, S, C, x_hbm, out_hbm, buf, acc, obuf):
    ngrp = C // _L
    nchunk = S // _CHUNK
    bpw = B // 32
    wid = lax.axis_index("s") * 2 + lax.axis_index("c")

    neg = jnp.full((_L,), -jnp.inf, dtype=jnp.float32)
    for bi in range(bpw):
        b = wid * bpw + bi
        for slot in range(ngrp * _K):
            acc[pl.ds(slot * _L, _L)] = neg

        def chunk_body(ck, carry):
            pltpu.sync_copy(
                x_hbm.at[b, pl.ds(ck * _CHUNK * C, _CHUNK * C)], buf)
            for g in range(ngrp):
                t = tuple(
                    acc[pl.ds((g * _K + j) * _L, _L)] for j in range(_K))

                def row_body(r, t):
                    t = list(t)
                    for u in range(_RU):
                        base = (r * _RU + u) * C + g * _L
                        v = buf[pl.ds(base, _L)]
                        for j in range(_K):
                            hi = jnp.maximum(t[j], v)
                            v = jnp.minimum(t[j], v)
                            t[j] = hi
                    return tuple(t)

                t = lax.fori_loop(0, _CHUNK // _RU, row_body, t)
                for j in range(_K):
                    acc[pl.ds((g * _K + j) * _L, _L)] = t[j]
            return carry

        lax.fori_loop(0, nchunk, chunk_body, 0)

        for g in range(ngrp):
            s = acc[pl.ds(g * _K * _L, _L)]
            for j in range(1, _K):
                s = s + acc[pl.ds((g * _K + j) * _L, _L)]
            obuf[pl.ds(g * _L, _L)] = s * jnp.float32(1.0 / _K)
        pltpu.sync_copy(obuf, out_hbm.at[b])


def kernel(x):
    b, s, c = x.shape
    xf = x.reshape(b, s * c)
    mesh = plsc.VectorSubcoreMesh(core_axis_name="c", subcore_axis_name="s")
    body = functools.partial(_sc_kernel_body, b, s, c)
    f = functools.partial(
        pl.kernel,
        mesh=mesh,
        out_type=jax.ShapeDtypeStruct((b, c), jnp.float32),
        scratch_types=[
            pltpu.VMEM((_CHUNK * c,), jnp.float32),
            pltpu.VMEM(((c // _L) * _K * _L,), jnp.float32),
            pltpu.VMEM((c,), jnp.float32),
        ],
    )(body)
    return f(xf)


# speedup vs baseline: 2.1999x; 2.1999x over previous
import functools
import jax
import jax.numpy as jnp
from jax import lax
from jax.experimental import pallas as pl
from jax.experimental.pallas import tpu as pltpu
from jax.experimental.pallas import tpu_sc as plsc

_K = 8


_UNROLL = 8
_NB = 4

_SORT8_NET = (
    (0, 2), (1, 3), (4, 6), (5, 7),
    (0, 4), (1, 5), (2, 6), (3, 7),
    (0, 1), (2, 3), (4, 5), (6, 7),
    (2, 4), (3, 5),
    (1, 4), (3, 6),
    (1, 2), (3, 4), (5, 6),
)


def _sort8_desc(vs):
    vs = list(vs)
    for i, j in _SORT8_NET:
        hi = jnp.maximum(vs[i], vs[j])
        lo = jnp.minimum(vs[i], vs[j])
        vs[i], vs[j] = hi, lo
    return vs


def _merge_top8(l, r):
    return [jnp.maximum(l[j], r[7 - j]) for j in range(8)]


def _bitonic_sort8(m):
    for d in (4, 2, 1):
        nm = list(m)
        for j in range(8):
            if (j % (2 * d)) < d:
                nm[j] = jnp.maximum(m[j], m[j + d])
                nm[j + d] = jnp.minimum(m[j], m[j + d])
        m = nm
    return m


def _tc_body(x_ref, o_ref):
    nvreg = x_ref.shape[1]
    c = x_ref.shape[3]
    init = jnp.full((8, c), -jnp.inf, jnp.float32)

    for bb in range(_NB):
        def step(i, carry):
            ta, tb = carry
            va = [x_ref[bb, i * 2 * _UNROLL + u] for u in range(_UNROLL)]
            vb = [x_ref[bb, i * 2 * _UNROLL + _UNROLL + u]
                  for u in range(_UNROLL)]
            sa = _sort8_desc(va)
            sb = _sort8_desc(vb)
            ta = tuple(_bitonic_sort8(_merge_top8(list(ta), sa)))
            tb = tuple(_bitonic_sort8(_merge_top8(list(tb), sb)))
            return (ta, tb)

        t0 = tuple([init] * 8)
        ta, tb = lax.fori_loop(0, nvreg // (2 * _UNROLL), step, (t0, t0))
        t = _bitonic_sort8(_merge_top8(list(ta), list(tb)))
        for d in (4, 2):
            r = [pltpu.roll(a, d, 0) for a in t]
            t = _bitonic_sort8(_merge_top8(t, r))
        r = [pltpu.roll(a, 1, 0) for a in t]
        m = _merge_top8(t, r)
        s = m[0]
        for j in range(1, 8):
            s = s + m[j]
        s = s * jnp.float32(1.0 / _K)
        o_ref[bb, :, :] = s[0:1, :]


def _tc_kernel(x):
    b, s, c = x.shape
    xr = x.reshape(b, s // 8, 8, c)
    out = pl.pallas_call(
        _tc_body,
        grid=(b // _NB,),
        in_specs=[pl.BlockSpec((_NB, s // 8, 8, c), lambda i: (i, 0, 0, 0))],
        out_specs=pl.BlockSpec((_NB, 1, c), lambda i: (i, 0, 0)),
        out_shape=jax.ShapeDtypeStruct((b, 1, c), jnp.float32),
    )(xr)
    return out.reshape(b, c)


_L = 16
_CHUNK = 512
_RU = 4
_NQ = 4


def _sc_body(B, S, C, x_hbm, out_hbm, buf, acc, mbuf, obuf, shared):
    ngrp = C // _L
    core = lax.axis_index("c")
    sid = lax.axis_index("s")
    b = core * (B // 2) + sid // _NQ
    q = sid % _NQ
    srows = S // _NQ
    neg = jnp.full((_L,), -jnp.inf, dtype=jnp.float32)

    for slot in range(ngrp * _K):
        acc[pl.ds(slot * _L, _L)] = neg

    def chunk_body(ck, carry):
        pltpu.sync_copy(
            x_hbm.at[b, pl.ds((q * srows + ck * _CHUNK) * C, _CHUNK * C)],
            buf)
        for g in range(ngrp):
            t = tuple(acc[pl.ds((g * _K + j) * _L, _L)] for j in range(_K))

            def row_body(r, t):
                t = list(t)
                for u in range(_RU):
                    base = (r * _RU + u) * C + g * _L
                    v = buf[pl.ds(base, _L)]
                    for j in range(_K):
                        hi = jnp.maximum(t[j], v)
                        v = jnp.minimum(t[j], v)
                        t[j] = hi
                return tuple(t)

            t = lax.fori_loop(0, _CHUNK // _RU, row_body, t)
            for j in range(_K):
                acc[pl.ds((g * _K + j) * _L, _L)] = t[j]
        return carry

    lax.fori_loop(0, srows // _CHUNK, chunk_body, 0)

    pltpu.sync_copy(acc, shared.at[sid])
    plsc.subcore_barrier()

    @pl.when(q == 0)
    def _merge_and_write():
        for sib in range(1, _NQ):
            pltpu.sync_copy(shared.at[sid + sib], mbuf)
            for g in range(ngrp):
                t = tuple(
                    acc[pl.ds((g * _K + j) * _L, _L)] for j in range(_K))
                t = list(t)
                for jj in range(_K):
                    v = mbuf[pl.ds((g * _K + jj) * _L, _L)]
                    for j in range(_K):
                        hi = jnp.maximum(t[j], v)
                        v = jnp.minimum(t[j], v)
                        t[j] = hi
                for j in range(_K):
                    acc[pl.ds((g * _K + j) * _L, _L)] = t[j]
        for g in range(ngrp):
            s = acc[pl.ds(g * _K * _L, _L)]
            for j in range(1, _K):
                s = s + acc[pl.ds((g * _K + j) * _L, _L)]
            obuf[pl.ds(g * _L, _L)] = s * jnp.float32(1.0 / _K)
        pltpu.sync_copy(obuf, out_hbm.at[b])


def _sc_kernel(x):
    b, s, c = x.shape
    xf = x.reshape(b, s * c)
    mesh = plsc.VectorSubcoreMesh(core_axis_name="c", subcore_axis_name="s")
    body = functools.partial(_sc_body, b, s, c)
    nacc = (c // _L) * _K * _L
    f = functools.partial(
        pl.kernel,
        mesh=mesh,
        out_type=jax.ShapeDtypeStruct((b, c), jnp.float32),
        scratch_types=[
            pltpu.VMEM((_CHUNK * c,), jnp.float32),
            pltpu.VMEM((nacc,), jnp.float32),
            pltpu.VMEM((nacc,), jnp.float32),
            pltpu.VMEM((c,), jnp.float32),
            pltpu.VMEM_SHARED((16, nacc), jnp.float32),
        ],
    )(body)
    return f(xf)


def kernel(x):
    b = x.shape[0]
    b_sc = b // 8
    out_tc = _tc_kernel(lax.slice_in_dim(x, 0, b - b_sc, axis=0))
    out_sc = _sc_kernel(lax.slice_in_dim(x, b - b_sc, b, axis=0))
    return jnp.concatenate([out_tc, out_sc], axis=0)


# speedup vs baseline: 6.3804x; 2.9003x over previous
import jax
import jax.numpy as jnp
from jax import lax
from jax.experimental import pallas as pl
from jax.experimental.pallas import tpu as pltpu

_K = 8
_UNROLL = 8


_SORT8_NET = (
    (0, 2), (1, 3), (4, 6), (5, 7),
    (0, 4), (1, 5), (2, 6), (3, 7),
    (0, 1), (2, 3), (4, 5), (6, 7),
    (2, 4), (3, 5),
    (1, 4), (3, 6),
    (1, 2), (3, 4), (5, 6),
)


def _sort8_desc(vs):
    vs = list(vs)
    for i, j in _SORT8_NET:
        hi = jnp.maximum(vs[i], vs[j])
        lo = jnp.minimum(vs[i], vs[j])
        vs[i], vs[j] = hi, lo
    return vs


def _merge_top8(l, r):
    return [jnp.maximum(l[j], r[7 - j]) for j in range(8)]


def _bitonic_sort8(m):
    for d in (4, 2, 1):
        nm = list(m)
        for j in range(8):
            if (j % (2 * d)) < d:
                nm[j] = jnp.maximum(m[j], m[j + d])
                nm[j + d] = jnp.minimum(m[j], m[j + d])
        m = nm
    return m


_NB = 4


def _body(x_ref, o_ref):
    nvreg = x_ref.shape[1]
    c = x_ref.shape[3]
    init = jnp.full((8, c), -jnp.inf, jnp.float32)

    ngrp = 2
    for bb in range(_NB):
        def step(i, carry):
            out = []
            for g, tg in enumerate(carry):
                vg = [x_ref[bb, (i * ngrp + g) * _UNROLL + u]
                      for u in range(_UNROLL)]
                sg = _sort8_desc(vg)
                out.append(tuple(_bitonic_sort8(_merge_top8(list(tg), sg))))
            return tuple(out)

        t0 = tuple([init] * 8)
        sets = lax.fori_loop(0, nvreg // (ngrp * _UNROLL), step, (t0,) * ngrp)
        t = list(sets[0])
        for g in range(1, ngrp):
            t = _bitonic_sort8(_merge_top8(t, list(sets[g])))
        for d in (4, 2):
            r = [pltpu.roll(a, d, 0) for a in t]
            t = _bitonic_sort8(_merge_top8(t, r))
        r = [pltpu.roll(a, 1, 0) for a in t]
        m = _merge_top8(t, r)
        s = m[0]
        for j in range(1, 8):
            s = s + m[j]
        s = s * jnp.float32(1.0 / _K)
        o_ref[bb, :, :] = s[0:1, :]


def kernel(x):
    b, s, c = x.shape
    xr = x.reshape(b, s // 8, 8, c)
    out = pl.pallas_call(
        _body,
        grid=(b // _NB,),
        in_specs=[pl.BlockSpec((_NB, s // 8, 8, c), lambda i: (i, 0, 0, 0))],
        out_specs=pl.BlockSpec((_NB, 1, c), lambda i: (i, 0, 0)),
        out_shape=jax.ShapeDtypeStruct((b, 1, c), jnp.float32),
    )(xr)
    return out.reshape(b, c)


# speedup vs baseline: 6.6394x; 1.0406x over previous
import jax
import jax.numpy as jnp
from jax import lax
from jax.experimental import pallas as pl
from jax.experimental.pallas import tpu as pltpu

_K = 8
_UNROLL = 8


_SORT8_NET = (
    (0, 2), (1, 3), (4, 6), (5, 7),
    (0, 4), (1, 5), (2, 6), (3, 7),
    (0, 1), (2, 3), (4, 5), (6, 7),
    (2, 4), (3, 5),
    (1, 4), (3, 6),
    (1, 2), (3, 4), (5, 6),
)


def _sort8_desc(vs):
    vs = list(vs)
    for i, j in _SORT8_NET:
        hi = jnp.maximum(vs[i], vs[j])
        lo = jnp.minimum(vs[i], vs[j])
        vs[i], vs[j] = hi, lo
    return vs


def _merge_top8(l, r):
    return [jnp.maximum(l[j], r[7 - j]) for j in range(8)]


def _bitonic_sort8(m):
    for d in (4, 2, 1):
        nm = list(m)
        for j in range(8):
            if (j % (2 * d)) < d:
                nm[j] = jnp.maximum(m[j], m[j + d])
                nm[j + d] = jnp.minimum(m[j], m[j + d])
        m = nm
    return m


_NB = 4


def _body(x_ref, o_ref):
    nvreg = x_ref.shape[1]
    c = x_ref.shape[3]
    init = jnp.full((8, c), -jnp.inf, jnp.float32)

    for bb in range(_NB):
        def step(i, carry):
            ta, tb = carry
            va = [x_ref[bb, i * 2 * _UNROLL + u] for u in range(_UNROLL)]
            vb = [x_ref[bb, i * 2 * _UNROLL + _UNROLL + u]
                  for u in range(_UNROLL)]
            sa = _sort8_desc(va)
            sb = _sort8_desc(vb)
            ta = tuple(_bitonic_sort8(_merge_top8(list(ta), sa)))
            tb = tuple(_bitonic_sort8(_merge_top8(list(tb), sb)))
            return (ta, tb)

        t0 = tuple([init] * 8)
        ta, tb = lax.fori_loop(0, nvreg // (2 * _UNROLL), step, (t0, t0))
        t = _bitonic_sort8(_merge_top8(list(ta), list(tb)))
        for d in (4, 2):
            r = [pltpu.roll(a, d, 0) for a in t]
            t = _bitonic_sort8(_merge_top8(t, r))
        r = [pltpu.roll(a, 1, 0) for a in t]
        m = _merge_top8(t, r)
        s = m[0]
        for j in range(1, 8):
            s = s + m[j]
        s = s * jnp.float32(1.0 / _K)
        o_ref[bb, :, :] = s[0:1, :]


def kernel(x):
    b, s, c = x.shape
    xr = x.reshape(b, s // 8, 8, c)
    out = pl.pallas_call(
        _body,
        grid=(b // _NB,),
        in_specs=[pl.BlockSpec((_NB, s // 8, 8, c), lambda i: (i, 0, 0, 0))],
        out_specs=pl.BlockSpec((_NB, 1, c), lambda i: (i, 0, 0)),
        out_shape=jax.ShapeDtypeStruct((b, 1, c), jnp.float32),
    )(xr)
    return out.reshape(b, c)
